# baseline (speedup 1.0000x reference)
import functools

import jax
import jax.numpy as jnp
from jax import lax
from jax.experimental import pallas as pl
from jax.experimental.pallas import tpu as pltpu
from jax.experimental.pallas import tpu_sc as plsc

_NC = 2
_NS = 16
_CHUNK = 128
_G = _CHUNK // 128

_ROW_BLK = 2000


def _mm3_body(x_ref, w_ref, self_ref, g0_ref, g1_ref):
    y = jnp.dot(x_ref[...], w_ref[...], preferred_element_type=jnp.float32)
    d = y.shape[1] // 2
    self_ref[...] = y[:, :d]
    h = d // 2
    g0_ref[...] = y[:, d:d + h]
    g1_ref[...] = y[:, d + h:]


def _mm3(x, w_cat):
    n, k = x.shape
    m = w_cat.shape[1]
    d = m // 2
    h = d // 2
    grid = n // _ROW_BLK
    row = lambda i: (i, 0)
    return pl.pallas_call(
        _mm3_body,
        grid=(grid,),
        in_specs=[
            pl.BlockSpec((_ROW_BLK, k), row),
            pl.BlockSpec((k, m), lambda i: (0, 0)),
        ],
        out_specs=[pl.BlockSpec((_ROW_BLK, d), row)] +
                  [pl.BlockSpec((_ROW_BLK, h), row)] * 2,
        out_shape=[jax.ShapeDtypeStruct((n, d), jnp.float32)] +
                  [jax.ShapeDtypeStruct((n, h), jnp.float32)] * 2,
    )(x, w_cat)


def _elu(x):
    return jnp.where(x > 0, x, jnp.exp(x) - 1.0)


def _att_core(self_ref, nb0_ref, nb1_ref, wq_ref, wk_ref, wa_ref, b_ref, do_elu):
    s = self_ref[...]
    nb = jnp.concatenate([nb0_ref[...], nb1_ref[...]], axis=1)
    wa = wa_ref[...]
    d = s.shape[1]
    u = jnp.dot(wk_ref[...], wa[:d], preferred_element_type=jnp.float32)
    v = jnp.dot(wq_ref[...], wa[d:], preferred_element_type=jnp.float32)
    qa = jnp.dot(s, v, preferred_element_type=jnp.float32)
    e_s = _elu(jnp.dot(s, u, preferred_element_type=jnp.float32) + qa)
    e_n = _elu(jnp.dot(nb, u, preferred_element_type=jnp.float32) + qa)
    m = jnp.maximum(e_s, e_n)
    p_s = jnp.exp(e_s - m)
    p_n = jnp.exp(e_n - m)
    z = p_s + p_n
    a_s = p_s / z
    a_n = p_n / z
    out = a_s * s + a_n * nb + b_ref[...]
    if do_elu:
        out = _elu(out)
    return out, jnp.concatenate([a_s, a_n], axis=1)


def _att_body(self_ref, nb0_ref, nb1_ref, wq_ref, wk_ref, wa_ref, b_ref,
              o_ref, att_ref, *, do_elu):
    out, att = _att_core(self_ref, nb0_ref, nb1_ref, wq_ref, wk_ref, wa_ref,
                         b_ref, do_elu)
    o_ref[...] = out
    att_ref[...] = att


def _att_head_body(self_ref, nb0_ref, nb1_ref, wq_ref, wk_ref, wa_ref, b_ref,
                   wc_ref, bc_ref, o_ref, att_ref, log_ref):
    out, att = _att_core(self_ref, nb0_ref, nb1_ref, wq_ref, wk_ref, wa_ref,
                         b_ref, False)
    o_ref[...] = out
    att_ref[...] = att
    log_ref[...] = jnp.dot(out, wc_ref[...], preferred_element_type=jnp.float32) \
        + bc_ref[...]


def _att_mm_body(self_ref, nb0_ref, nb1_ref, wq_ref, wk_ref, wa_ref, b_ref,
                 w2_ref, self2_ref, g20_ref, g21_ref, att_ref):
    out, att = _att_core(self_ref, nb0_ref, nb1_ref, wq_ref, wk_ref, wa_ref,
                         b_ref, True)
    y = jnp.dot(out, w2_ref[...], preferred_element_type=jnp.float32)
    d = y.shape[1] // 2
    h = d // 2
    self2_ref[...] = y[:, :d]
    g20_ref[...] = y[:, d:d + h]
    g21_ref[...] = y[:, d + h:]
    att_ref[...] = att


def _att_mm(self_ft, nbq, wq, wk, wa, bias, w2_cat):
    n, d = self_ft.shape
    m = w2_cat.shape[1]
    h = m // 4
    grid = n // _ROW_BLK
    row = lambda i: (i, 0)
    fix = lambda i: (0, 0)
    return pl.pallas_call(
        _att_mm_body,
        grid=(grid,),
        in_specs=[pl.BlockSpec((_ROW_BLK, d), row)] +
                 [pl.BlockSpec((_ROW_BLK, d // 2), row)] * 2 +
                 [pl.BlockSpec(w.shape, fix) for w in (wq, wk, wa, bias, w2_cat)],
        out_specs=[
            pl.BlockSpec((_ROW_BLK, m // 2), row),
            pl.BlockSpec((_ROW_BLK, h), row),
            pl.BlockSpec((_ROW_BLK, h), row),
            pl.BlockSpec((_ROW_BLK, 2), row),
        ],
        out_shape=[
            jax.ShapeDtypeStruct((n, m // 2), jnp.float32),
            jax.ShapeDtypeStruct((n, h), jnp.float32),
            jax.ShapeDtypeStruct((n, h), jnp.float32),
            jax.ShapeDtypeStruct((n, 2), jnp.float32),
        ],
    )(self_ft, *nbq, wq, wk, wa, bias, w2_cat)


def _att(self_ft, nbq, wq, wk, wa, bias, do_elu):
    n, d = self_ft.shape
    grid = n // _ROW_BLK
    row = lambda i: (i, 0)
    fix = lambda i: (0, 0)
    return pl.pallas_call(
        functools.partial(_att_body, do_elu=do_elu),
        grid=(grid,),
        in_specs=[pl.BlockSpec((_ROW_BLK, d), row)] +
                 [pl.BlockSpec((_ROW_BLK, d // 2), row)] * 2 +
                 [pl.BlockSpec(w.shape, fix) for w in (wq, wk, wa, bias)],
        out_specs=[
            pl.BlockSpec((_ROW_BLK, d), row),
            pl.BlockSpec((_ROW_BLK, 2), row),
        ],
        out_shape=[
            jax.ShapeDtypeStruct((n, d), jnp.float32),
            jax.ShapeDtypeStruct((n, 2), jnp.float32),
        ],
    )(self_ft, *nbq, wq, wk, wa, bias)


def _att_head(self_ft, nbq, wq, wk, wa, bias, wc, bc):
    n, d = self_ft.shape
    ncls = wc.shape[1]
    grid = n // _ROW_BLK
    row = lambda i: (i, 0)
    fix = lambda i: (0, 0)
    return pl.pallas_call(
        _att_head_body,
        grid=(grid,),
        in_specs=[pl.BlockSpec((_ROW_BLK, d), row)] +
                 [pl.BlockSpec((_ROW_BLK, d // 2), row)] * 2 +
                 [pl.BlockSpec(w.shape, fix) for w in (wq, wk, wa, bias, wc, bc)],
        out_specs=[
            pl.BlockSpec((_ROW_BLK, d), row),
            pl.BlockSpec((_ROW_BLK, 2), row),
            pl.BlockSpec((_ROW_BLK, ncls), row),
        ],
        out_shape=[
            jax.ShapeDtypeStruct((n, d), jnp.float32),
            jax.ShapeDtypeStruct((n, 2), jnp.float32),
            jax.ShapeDtypeStruct((n, ncls), jnp.float32),
        ],
    )(self_ft, *nbq, wq, wk, wa, bias, wc, bc)


def _spmm_sc_call(n_pad, e_pad):
    rows_per_tile = e_pad // 128 // _NS
    n_chunks = rows_per_tile // _G
    assert n_chunks % 5 == 0 and n_chunks >= 10
    n_iters = n_chunks // 5
    npt = n_pad // _NS
    nzf = npt // _CHUNK
    nzr = npt - nzf * _CHUNK
    assert nzr % 8 == 0

    mesh = plsc.VectorSubcoreMesh(core_axis_name="c", subcore_axis_name="s",
                                  num_cores=_NC, num_subcores=_NS)

    @functools.partial(
        pl.kernel,
        out_type=[
            jax.ShapeDtypeStruct((n_pad, 32), jnp.float32),
            jax.ShapeDtypeStruct((n_pad, 32), jnp.float32),
        ],
        mesh=mesh,
        compiler_params=pltpu.CompilerParams(use_tc_tiling_on_sc=False),
        scratch_types=[
            pltpu.VMEM((5, _G, 128), jnp.int32),
            pltpu.VMEM((5, _G, 128), jnp.int32),
            pltpu.VMEM((5, _CHUNK), jnp.float32),
            pltpu.VMEM((5, _CHUNK, 32), jnp.float32),
            pltpu.VMEM_SHARED((n_pad, 32), jnp.float32),
        ] + [pltpu.SemaphoreType.DMA] * 15,
    )
    def spmm_k(g0_hbm, g1_hbm, src_hbm, dst_hbm, vals_hbm, out0_hbm, out1_hbm,
               srcv, dstv, valv, rows, acc,
               gs0, gs1, gs2, gs3, gs4, ss0, ss1, ss2, ss3, ss4,
               is0, is1, is2, is3, is4):
        gsem = [gs0, gs1, gs2, gs3, gs4]
        ssem = [ss0, ss1, ss2, ss3, ss4]
        isem = [is0, is1, is2, is3, is4]
        c = lax.axis_index("c")
        s = lax.axis_index("s")

        def fire_idx(k, slot):
            base = s * rows_per_tile + k * _G
            pltpu.async_copy(src_hbm.at[pl.ds(base, _G)], srcv.at[slot], isem[slot])
            pltpu.async_copy(dst_hbm.at[pl.ds(base, _G)], dstv.at[slot], isem[slot])
            pltpu.async_copy(vals_hbm.at[pl.ds(base * 128, _CHUNK)],
                             valv.at[slot], isem[slot])

        def wait_idx(slot):
            pltpu.make_async_copy(src_hbm.at[pl.ds(0, _G)], srcv.at[slot],
                                  isem[slot]).wait()
            pltpu.make_async_copy(dst_hbm.at[pl.ds(0, _G)], dstv.at[slot],
                                  isem[slot]).wait()
            pltpu.make_async_copy(vals_hbm.at[pl.ds(0, _CHUNK)], valv.at[slot],
                                  isem[slot]).wait()

        def fire_gathers(slot):
            @pl.when(c == 0)
            def _():
                for j in range(_G):
                    pltpu.async_copy(g0_hbm.at[srcv.at[slot, j]],
                                     rows.at[slot, pl.ds(j * 128, 128)],
                                     gsem[slot])

            @pl.when(c == 1)
            def _():
                for j in range(_G):
                    pltpu.async_copy(g1_hbm.at[srcv.at[slot, j]],
                                     rows.at[slot, pl.ds(j * 128, 128)],
                                     gsem[slot])

        def wait_gathers(slot):
            for j in range(_G):
                pltpu.make_async_copy(g0_hbm.at[srcv.at[slot, j]],
                                      rows.at[slot, pl.ds(j * 128, 128)],
                                      gsem[slot]).wait()

        lane = [jnp.full((16,), i, jnp.int32) for i in range(16)]
        dnums = lax.GatherDimensionNumbers(
            offset_dims=(), collapsed_slice_dims=(0,), start_index_map=(0,))

        def scale_scatter(slot):
            for j in range(_G):
                def sgrp(g, carry):
                    r0 = j * 128 + g * 16
                    v16 = valv[slot, pl.ds(r0, 16)]
                    for k2 in range(16):
                        vb = lax.gather(
                            v16, lane[k2][:, None], dnums, (1,),
                            mode=lax.GatherScatterMode.PROMISE_IN_BOUNDS)
                        rows[slot, r0 + k2, 0:16] = rows[slot, r0 + k2, 0:16] * vb
                        rows[slot, r0 + k2, 16:32] = rows[slot, r0 + k2, 16:32] * vb
                    return carry
                lax.fori_loop(0, 8, sgrp, 0)
                pltpu.async_copy(rows.at[slot, pl.ds(j * 128, 128)],
                                 acc.at[dstv.at[slot, j]], ssem[slot], add=True)

        def drain_scatter(slot):
            for j in range(_G):
                pltpu.make_async_copy(rows.at[slot, pl.ds(j * 128, 128)],
                                      acc.at[dstv.at[slot, j]],
                                      ssem[slot]).wait()

        def zfill(i, carry):
            rows[0, i, 0:16] = jnp.zeros((16,), jnp.float32)
            rows[0, i, 16:32] = jnp.zeros((16,), jnp.float32)
            return carry
        lax.fori_loop(0, _CHUNK, zfill, 0)

        def zcopy(q, carry):
            pltpu.sync_copy(rows.at[0], acc.at[pl.ds(s * npt + q * _CHUNK, _CHUNK)])
            return carry
        lax.fori_loop(0, nzf, zcopy, 0)
        if nzr:
            pltpu.sync_copy(rows.at[0, pl.ds(0, nzr)],
                            acc.at[pl.ds(s * npt + nzf * _CHUNK, nzr)])
        plsc.subcore_barrier()

        fire_idx(0, 0)
        fire_idx(1, 1)
        fire_idx(2, 2)
        wait_idx(0)
        fire_gathers(0)
        wait_idx(1)
        fire_gathers(1)

        def body(t, carry):
            for p in range(5):
                k = 5 * t + p
                wait_gathers(p)
                scale_scatter(p)
                sl3 = (p + 3) % 5
                if p >= 2:
                    drain_scatter(sl3)
                    @pl.when(t < n_iters - 1)
                    def _():
                        fire_idx(k + 3, sl3)
                else:
                    @pl.when(t > 0)
                    def _():
                        drain_scatter(sl3)
                    fire_idx(k + 3, sl3)
                sl2 = (p + 2) % 5
                if p <= 2:
                    wait_idx(sl2)
                    fire_gathers(sl2)
                else:
                    @pl.when(t < n_iters - 1)
                    def _():
                        wait_idx(sl2)
                        fire_gathers(sl2)
            return carry
        lax.fori_loop(0, n_iters, body, 0)
        drain_scatter(3)
        drain_scatter(4)
        plsc.subcore_barrier()

        @pl.when(c == 0)
        def _():
            pltpu.sync_copy(acc.at[pl.ds(s * npt, npt)],
                            out0_hbm.at[pl.ds(s * npt, npt)])

        @pl.when(c == 1)
        def _():
            pltpu.sync_copy(acc.at[pl.ds(s * npt, npt)],
                            out1_hbm.at[pl.ds(s * npt, npt)])

    return spmm_k


def _spmm(gq, src2, dst2, vals2):
    n = gq[0].shape[0]
    n_pad = ((n + 8 * _NS - 1) // (8 * _NS)) * (8 * _NS)
    e_pad = src2.shape[0] * 128
    return list(_spmm_sc_call(n_pad, e_pad)(gq[0], gq[1], src2, dst2, vals2))


def _prep_edges(idx, vals):
    e = vals.shape[0]
    unit = _NS * _CHUNK * 5
    e_pad = ((e + unit - 1) // unit) * unit
    pad = e_pad - e
    src = jnp.concatenate([idx[1].astype(jnp.int32),
                           jnp.zeros((pad,), jnp.int32)])
    dst = jnp.concatenate([idx[0].astype(jnp.int32),
                           jnp.zeros((pad,), jnp.int32)])
    v = jnp.concatenate([vals, jnp.zeros((pad,), jnp.float32)])
    return (src.reshape(e_pad // 128, 128), dst.reshape(e_pad // 128, 128), v)


def kernel(ft_p, ft_a, adj_pa_index, adj_pa_vals, adj_ap_index, adj_ap_vals,
           l1_p_w_self, l1_p_w_rel_a, l1_p_bias, l1_p_w_query, l1_p_w_keys, l1_p_w_att,
           l1_a_w_self, l1_a_w_rel_p, l1_a_bias, l1_a_w_query, l1_a_w_keys, l1_a_w_att,
           l2_p_w_self, l2_p_w_rel_a, l2_p_bias, l2_p_w_query, l2_p_w_keys, l2_p_w_att,
           l2_a_w_self, l2_a_w_rel_p, l2_a_bias, l2_a_w_query, l2_a_w_keys, l2_a_w_att,
           embd2class_p, cls_bias_p):
    pa_src, pa_dst, pa_vals = _prep_edges(adj_pa_index, adj_pa_vals)
    ap_src, ap_dst, ap_vals = _prep_edges(adj_ap_index, adj_ap_vals)

    w1p = jnp.concatenate([l1_p_w_self, l1_a_w_rel_p], axis=1)
    w1a = jnp.concatenate([l1_a_w_self, l1_p_w_rel_a], axis=1)
    self1_p, *gap = _mm3(ft_p, w1p)
    self1_a, *gpa = _mm3(ft_a, w1a)

    nbp = _spmm(gpa, pa_src, pa_dst, pa_vals)
    nba = _spmm(gap, ap_src, ap_dst, ap_vals)

    w2p = jnp.concatenate([l2_p_w_self, l2_a_w_rel_p], axis=1)
    w2a = jnp.concatenate([l2_a_w_self, l2_p_w_rel_a], axis=1)
    self2_p, g2ap0, g2ap1, att1_p = _att_mm(self1_p, nbp, l1_p_w_query,
                                            l1_p_w_keys, l1_p_w_att, l1_p_bias,
                                            w2p)
    self2_a, g2pa0, g2pa1, att1_a = _att_mm(self1_a, nba, l1_a_w_query,
                                            l1_a_w_keys, l1_a_w_att, l1_a_bias,
                                            w2a)
    g2ap = [g2ap0, g2ap1]
    g2pa = [g2pa0, g2pa1]

    nb2p = _spmm(g2pa, pa_src, pa_dst, pa_vals)
    nb2a = _spmm(g2ap, ap_src, ap_dst, ap_vals)

    x2_p, att2_p, logits_p = _att_head(self2_p, nb2p, l2_p_w_query,
                                       l2_p_w_keys, l2_p_w_att, l2_p_bias,
                                       embd2class_p, cls_bias_p)
    _, att2_a = _att(self2_a, nb2a, l2_a_w_query, l2_a_w_keys,
                     l2_a_w_att, l2_a_bias, do_elu=False)

    return (logits_p, x2_p, att1_p, att1_a, att2_p, att2_a)


# speedup vs baseline: 1.0517x; 1.0517x over previous
import functools

import jax
import jax.numpy as jnp
from jax import lax
from jax.experimental import pallas as pl
from jax.experimental.pallas import tpu as pltpu
from jax.experimental.pallas import tpu_sc as plsc

_NC = 2
_NS = 16
_CHUNK = 128
_G = _CHUNK // 128

_ROW_BLK = 2000


def _mm3_body(x_ref, w_ref, self_ref, g0_ref, g1_ref):
    y = jnp.dot(x_ref[...], w_ref[...], preferred_element_type=jnp.float32)
    d = y.shape[1] // 2
    self_ref[...] = y[:, :d]
    h = d // 2
    g0_ref[...] = y[:, d:d + h]
    g1_ref[...] = y[:, d + h:]


def _mm3(x, w_cat):
    n, k = x.shape
    m = w_cat.shape[1]
    d = m // 2
    h = d // 2
    grid = n // _ROW_BLK
    row = lambda i: (i, 0)
    return pl.pallas_call(
        _mm3_body,
        grid=(grid,),
        in_specs=[
            pl.BlockSpec((_ROW_BLK, k), row),
            pl.BlockSpec((k, m), lambda i: (0, 0)),
        ],
        out_specs=[pl.BlockSpec((_ROW_BLK, d), row)] +
                  [pl.BlockSpec((_ROW_BLK, h), row)] * 2,
        out_shape=[jax.ShapeDtypeStruct((n, d), jnp.float32)] +
                  [jax.ShapeDtypeStruct((n, h), jnp.float32)] * 2,
    )(x, w_cat)


def _elu(x):
    return jnp.where(x > 0, x, jnp.exp(x) - 1.0)


def _att_core(self_ref, nb0_ref, nb1_ref, wq_ref, wk_ref, wa_ref, b_ref, do_elu):
    s = self_ref[...]
    nb = jnp.concatenate([nb0_ref[...], nb1_ref[...]], axis=1)
    wa = wa_ref[...]
    d = s.shape[1]
    u = jnp.dot(wk_ref[...], wa[:d], preferred_element_type=jnp.float32)
    v = jnp.dot(wq_ref[...], wa[d:], preferred_element_type=jnp.float32)
    qa = jnp.dot(s, v, preferred_element_type=jnp.float32)
    e_s = _elu(jnp.dot(s, u, preferred_element_type=jnp.float32) + qa)
    e_n = _elu(jnp.dot(nb, u, preferred_element_type=jnp.float32) + qa)
    m = jnp.maximum(e_s, e_n)
    p_s = jnp.exp(e_s - m)
    p_n = jnp.exp(e_n - m)
    z = p_s + p_n
    a_s = p_s / z
    a_n = p_n / z
    out = a_s * s + a_n * nb + b_ref[...]
    if do_elu:
        out = _elu(out)
    return out, jnp.concatenate([a_s, a_n], axis=1)


def _att_body(self_ref, nb0_ref, nb1_ref, wq_ref, wk_ref, wa_ref, b_ref,
              o_ref, att_ref, *, do_elu):
    out, att = _att_core(self_ref, nb0_ref, nb1_ref, wq_ref, wk_ref, wa_ref,
                         b_ref, do_elu)
    o_ref[...] = out
    att_ref[...] = att


def _att_head_body(self_ref, nb0_ref, nb1_ref, wq_ref, wk_ref, wa_ref, b_ref,
                   wc_ref, bc_ref, o_ref, att_ref, log_ref):
    out, att = _att_core(self_ref, nb0_ref, nb1_ref, wq_ref, wk_ref, wa_ref,
                         b_ref, False)
    o_ref[...] = out
    att_ref[...] = att
    log_ref[...] = jnp.dot(out, wc_ref[...], preferred_element_type=jnp.float32) \
        + bc_ref[...]


def _att_mm_body(self_ref, nb0_ref, nb1_ref, wq_ref, wk_ref, wa_ref, b_ref,
                 w2_ref, self2_ref, g20_ref, g21_ref, att_ref):
    out, att = _att_core(self_ref, nb0_ref, nb1_ref, wq_ref, wk_ref, wa_ref,
                         b_ref, True)
    y = jnp.dot(out, w2_ref[...], preferred_element_type=jnp.float32)
    d = y.shape[1] // 2
    h = d // 2
    self2_ref[...] = y[:, :d]
    g20_ref[...] = y[:, d:d + h]
    g21_ref[...] = y[:, d + h:]
    att_ref[...] = att


def _att_mm(self_ft, nbq, wq, wk, wa, bias, w2_cat):
    n, d = self_ft.shape
    m = w2_cat.shape[1]
    h = m // 4
    grid = n // _ROW_BLK
    row = lambda i: (i, 0)
    fix = lambda i: (0, 0)
    return pl.pallas_call(
        _att_mm_body,
        grid=(grid,),
        in_specs=[pl.BlockSpec((_ROW_BLK, d), row)] +
                 [pl.BlockSpec((_ROW_BLK, d // 2), row)] * 2 +
                 [pl.BlockSpec(w.shape, fix) for w in (wq, wk, wa, bias, w2_cat)],
        out_specs=[
            pl.BlockSpec((_ROW_BLK, m // 2), row),
            pl.BlockSpec((_ROW_BLK, h), row),
            pl.BlockSpec((_ROW_BLK, h), row),
            pl.BlockSpec((_ROW_BLK, 2), row),
        ],
        out_shape=[
            jax.ShapeDtypeStruct((n, m // 2), jnp.float32),
            jax.ShapeDtypeStruct((n, h), jnp.float32),
            jax.ShapeDtypeStruct((n, h), jnp.float32),
            jax.ShapeDtypeStruct((n, 2), jnp.float32),
        ],
    )(self_ft, *nbq, wq, wk, wa, bias, w2_cat)


def _att(self_ft, nbq, wq, wk, wa, bias, do_elu):
    n, d = self_ft.shape
    grid = n // _ROW_BLK
    row = lambda i: (i, 0)
    fix = lambda i: (0, 0)
    return pl.pallas_call(
        functools.partial(_att_body, do_elu=do_elu),
        grid=(grid,),
        in_specs=[pl.BlockSpec((_ROW_BLK, d), row)] +
                 [pl.BlockSpec((_ROW_BLK, d // 2), row)] * 2 +
                 [pl.BlockSpec(w.shape, fix) for w in (wq, wk, wa, bias)],
        out_specs=[
            pl.BlockSpec((_ROW_BLK, d), row),
            pl.BlockSpec((_ROW_BLK, 2), row),
        ],
        out_shape=[
            jax.ShapeDtypeStruct((n, d), jnp.float32),
            jax.ShapeDtypeStruct((n, 2), jnp.float32),
        ],
    )(self_ft, *nbq, wq, wk, wa, bias)


def _att_head(self_ft, nbq, wq, wk, wa, bias, wc, bc):
    n, d = self_ft.shape
    ncls = wc.shape[1]
    grid = n // _ROW_BLK
    row = lambda i: (i, 0)
    fix = lambda i: (0, 0)
    return pl.pallas_call(
        _att_head_body,
        grid=(grid,),
        in_specs=[pl.BlockSpec((_ROW_BLK, d), row)] +
                 [pl.BlockSpec((_ROW_BLK, d // 2), row)] * 2 +
                 [pl.BlockSpec(w.shape, fix) for w in (wq, wk, wa, bias, wc, bc)],
        out_specs=[
            pl.BlockSpec((_ROW_BLK, d), row),
            pl.BlockSpec((_ROW_BLK, 2), row),
            pl.BlockSpec((_ROW_BLK, ncls), row),
        ],
        out_shape=[
            jax.ShapeDtypeStruct((n, d), jnp.float32),
            jax.ShapeDtypeStruct((n, 2), jnp.float32),
            jax.ShapeDtypeStruct((n, ncls), jnp.float32),
        ],
    )(self_ft, *nbq, wq, wk, wa, bias, wc, bc)


def _spmm_sc_call(n_pad, e_pad):
    rows_per_tile = e_pad // 128 // _NS
    n_chunks = rows_per_tile // _G
    assert n_chunks % 5 == 0 and n_chunks >= 10
    n_iters = n_chunks // 5
    npt = n_pad // _NS
    nzf = npt // _CHUNK
    nzr = npt - nzf * _CHUNK
    assert nzr % 8 == 0

    mesh = plsc.VectorSubcoreMesh(core_axis_name="c", subcore_axis_name="s",
                                  num_cores=_NC, num_subcores=_NS)

    @functools.partial(
        pl.kernel,
        out_type=[
            jax.ShapeDtypeStruct((n_pad, 32), jnp.float32),
            jax.ShapeDtypeStruct((n_pad, 32), jnp.float32),
        ],
        mesh=mesh,
        compiler_params=pltpu.CompilerParams(use_tc_tiling_on_sc=False),
        scratch_types=[
            pltpu.VMEM((5, _G, 128), jnp.int32),
            pltpu.VMEM((5, _G, 128), jnp.int32),
            pltpu.VMEM((5, _CHUNK), jnp.float32),
            pltpu.VMEM((5, _CHUNK, 32), jnp.float32),
            pltpu.VMEM_SHARED((n_pad, 32), jnp.float32),
        ] + [pltpu.SemaphoreType.DMA] * 15,
    )
    def spmm_k(g0_hbm, g1_hbm, src_hbm, dst_hbm, vals_hbm, out0_hbm, out1_hbm,
               srcv, dstv, valv, rows, acc,
               gs0, gs1, gs2, gs3, gs4, ss0, ss1, ss2, ss3, ss4,
               is0, is1, is2, is3, is4):
        gsem = [gs0, gs1, gs2, gs3, gs4]
        ssem = [ss0, ss1, ss2, ss3, ss4]
        isem = [is0, is1, is2, is3, is4]
        c = lax.axis_index("c")
        s = lax.axis_index("s")

        def fire_idx(k, slot):
            base = s * rows_per_tile + k * _G
            pltpu.async_copy(src_hbm.at[pl.ds(base, _G)], srcv.at[slot], isem[slot])
            pltpu.async_copy(dst_hbm.at[pl.ds(base, _G)], dstv.at[slot], isem[slot])
            pltpu.async_copy(vals_hbm.at[pl.ds(base * 128, _CHUNK)],
                             valv.at[slot], isem[slot])

        def wait_idx(slot):
            pltpu.make_async_copy(src_hbm.at[pl.ds(0, _G)], srcv.at[slot],
                                  isem[slot]).wait()
            pltpu.make_async_copy(dst_hbm.at[pl.ds(0, _G)], dstv.at[slot],
                                  isem[slot]).wait()
            pltpu.make_async_copy(vals_hbm.at[pl.ds(0, _CHUNK)], valv.at[slot],
                                  isem[slot]).wait()

        def fire_gathers(slot):
            @pl.when(c == 0)
            def _():
                for j in range(_G):
                    pltpu.async_copy(g0_hbm.at[srcv.at[slot, j]],
                                     rows.at[slot, pl.ds(j * 128, 128)],
                                     gsem[slot])

            @pl.when(c == 1)
            def _():
                for j in range(_G):
                    pltpu.async_copy(g1_hbm.at[srcv.at[slot, j]],
                                     rows.at[slot, pl.ds(j * 128, 128)],
                                     gsem[slot])

        def wait_gathers(slot):
            for j in range(_G):
                pltpu.make_async_copy(g0_hbm.at[srcv.at[slot, j]],
                                      rows.at[slot, pl.ds(j * 128, 128)],
                                      gsem[slot]).wait()

        lane = [jnp.full((16,), i, jnp.int32) for i in range(16)]
        dnums = lax.GatherDimensionNumbers(
            offset_dims=(), collapsed_slice_dims=(0,), start_index_map=(0,))

        def scale_scatter(slot):
            for j in range(_G):
                def sgrp(g, carry):
                    r0 = j * 128 + g * 16
                    v16 = valv[slot, pl.ds(r0, 16)]
                    for k2 in range(16):
                        vb = lax.gather(
                            v16, lane[k2][:, None], dnums, (1,),
                            mode=lax.GatherScatterMode.PROMISE_IN_BOUNDS)
                        rows[slot, r0 + k2, 0:16] = rows[slot, r0 + k2, 0:16] * vb
                        rows[slot, r0 + k2, 16:32] = rows[slot, r0 + k2, 16:32] * vb
                    return carry
                lax.fori_loop(0, 8, sgrp, 0)
                pltpu.async_copy(rows.at[slot, pl.ds(j * 128, 128)],
                                 acc.at[dstv.at[slot, j]], ssem[slot], add=True)

        def drain_scatter(slot):
            for j in range(_G):
                pltpu.make_async_copy(rows.at[slot, pl.ds(j * 128, 128)],
                                      acc.at[dstv.at[slot, j]],
                                      ssem[slot]).wait()

        def zfill(i, carry):
            rows[0, i, 0:16] = jnp.zeros((16,), jnp.float32)
            rows[0, i, 16:32] = jnp.zeros((16,), jnp.float32)
            return carry
        lax.fori_loop(0, _CHUNK, zfill, 0)

        def zcopy(q, carry):
            pltpu.sync_copy(rows.at[0], acc.at[pl.ds(s * npt + q * _CHUNK, _CHUNK)])
            return carry
        lax.fori_loop(0, nzf, zcopy, 0)
        if nzr:
            pltpu.sync_copy(rows.at[0, pl.ds(0, nzr)],
                            acc.at[pl.ds(s * npt + nzf * _CHUNK, nzr)])
        plsc.subcore_barrier()

        fire_idx(0, 0)
        fire_idx(1, 1)
        fire_idx(2, 2)
        wait_idx(0)
        fire_gathers(0)
        wait_idx(1)
        fire_gathers(1)

        def body(t, carry):
            for p in range(5):
                k = 5 * t + p
                wait_gathers(p)
                scale_scatter(p)
                sl3 = (p + 3) % 5
                if p >= 2:
                    drain_scatter(sl3)
                    @pl.when(t < n_iters - 1)
                    def _():
                        fire_idx(k + 3, sl3)
                else:
                    @pl.when(t > 0)
                    def _():
                        drain_scatter(sl3)
                    fire_idx(k + 3, sl3)
                sl2 = (p + 2) % 5
                if p <= 2:
                    wait_idx(sl2)
                    fire_gathers(sl2)
                else:
                    @pl.when(t < n_iters - 1)
                    def _():
                        wait_idx(sl2)
                        fire_gathers(sl2)
            return carry
        lax.fori_loop(0, n_iters, body, 0)
        drain_scatter(3)
        drain_scatter(4)
        plsc.subcore_barrier()

        @pl.when(c == 0)
        def _():
            pltpu.sync_copy(acc.at[pl.ds(s * npt, npt)],
                            out0_hbm.at[pl.ds(s * npt, npt)])

        @pl.when(c == 1)
        def _():
            pltpu.sync_copy(acc.at[pl.ds(s * npt, npt)],
                            out1_hbm.at[pl.ds(s * npt, npt)])

    return spmm_k


def _spmm(gq, src2, dst2, vals2):
    n = gq[0].shape[0]
    n_pad = ((n + 8 * _NS - 1) // (8 * _NS)) * (8 * _NS)
    e_pad = src2.shape[0] * 128
    return list(_spmm_sc_call(n_pad, e_pad)(gq[0], gq[1], src2, dst2, vals2))


def _prep_edges(idx, vals):
    e = vals.shape[0]
    unit = _NS * _CHUNK * 5
    e_pad = ((e + unit - 1) // unit) * unit
    pad = e_pad - e
    src = jnp.concatenate([idx[1].astype(jnp.int32),
                           jnp.zeros((pad,), jnp.int32)])
    dst = jnp.concatenate([idx[0].astype(jnp.int32),
                           jnp.zeros((pad,), jnp.int32)])
    v = jnp.concatenate([vals, jnp.zeros((pad,), jnp.float32)])
    return (src.reshape(e_pad // 128, 128), dst.reshape(e_pad // 128, 128), v)


def kernel(ft_p, ft_a, adj_pa_index, adj_pa_vals, adj_ap_index, adj_ap_vals,
           l1_p_w_self, l1_p_w_rel_a, l1_p_bias, l1_p_w_query, l1_p_w_keys, l1_p_w_att,
           l1_a_w_self, l1_a_w_rel_p, l1_a_bias, l1_a_w_query, l1_a_w_keys, l1_a_w_att,
           l2_p_w_self, l2_p_w_rel_a, l2_p_bias, l2_p_w_query, l2_p_w_keys, l2_p_w_att,
           l2_a_w_self, l2_a_w_rel_p, l2_a_bias, l2_a_w_query, l2_a_w_keys, l2_a_w_att,
           embd2class_p, cls_bias_p):
    pa_src, pa_dst, pa_vals = _prep_edges(adj_pa_index, adj_pa_vals)
    ap_src, ap_dst, ap_vals = _prep_edges(adj_ap_index, adj_ap_vals)

    w1p = jnp.concatenate([l1_p_w_self, l1_a_w_rel_p], axis=1)
    w1a = jnp.concatenate([l1_a_w_self, l1_p_w_rel_a], axis=1)
    self1_p, *gap = _mm3(ft_p, w1p)
    self1_a, *gpa = _mm3(ft_a, w1a)

    nbp = _spmm(gpa, pa_src, pa_dst, pa_vals)
    nba = _spmm(gap, ap_src, ap_dst, ap_vals)

    x1_p, att1_p = _att(self1_p, nbp, l1_p_w_query, l1_p_w_keys,
                        l1_p_w_att, l1_p_bias, do_elu=True)
    x1_a, att1_a = _att(self1_a, nba, l1_a_w_query, l1_a_w_keys,
                        l1_a_w_att, l1_a_bias, do_elu=True)

    w2p = jnp.concatenate([l2_p_w_self, l2_a_w_rel_p], axis=1)
    w2a = jnp.concatenate([l2_a_w_self, l2_p_w_rel_a], axis=1)
    self2_p, *g2ap = _mm3(x1_p, w2p)
    self2_a, *g2pa = _mm3(x1_a, w2a)

    nb2p = _spmm(g2pa, pa_src, pa_dst, pa_vals)
    nb2a = _spmm(g2ap, ap_src, ap_dst, ap_vals)

    x2_p, att2_p, logits_p = _att_head(self2_p, nb2p, l2_p_w_query,
                                       l2_p_w_keys, l2_p_w_att, l2_p_bias,
                                       embd2class_p, cls_bias_p)
    _, att2_a = _att(self2_a, nb2a, l2_a_w_query, l2_a_w_keys,
                     l2_a_w_att, l2_a_bias, do_elu=False)

    return (logits_p, x2_p, att1_p, att1_a, att2_p, att2_a)
